# Initial kernel scaffold; baseline (speedup 1.0000x reference)
import jax
import jax.numpy as jnp
from jax.experimental import pallas as pl


def kernel(x, edge_index, batch, batch_size, W1, b1, gamma, beta, W2, b2, mW1, mb1, mW2, mb2):
    raise NotImplementedError("write your pallas kernel here")


# speedup vs baseline: 4.6242x; 4.6242x over previous
import functools

import jax
import jax.numpy as jnp
from jax import lax
from jax.experimental import pallas as pl
from jax.experimental.pallas import tpu as pltpu
from jax.experimental.pallas import tpu_sc as plsc

N = 10000
E = 320000
D = 128
H = 128
O = 64
B = 128
L = 3

NC = 2
NS = 16
NW = NC * NS
EPW = E // NW
CHUNK = 80
NCHUNK = EPW // CHUNK
NPAD = 10240
ZROWS = NPAD // NS
ZBUF = 128


def _segsum_body(h_hbm, src_hbm, dst_hbm, out_hbm,
                 src_v, dst_v, rows_v, zbuf_v, agg_sh, sem):
    cid = lax.axis_index("c")
    sid = lax.axis_index("s")

    def zero_body(i, _):
        r = i // (D // 16)
        c = (i % (D // 16)) * 16
        zbuf_v[r, pl.ds(c, 16)] = jnp.zeros((16,), jnp.float32)
        return 0
    lax.fori_loop(0, ZBUF * (D // 16), zero_body, 0)

    def zcopy_body(i, _):
        pltpu.sync_copy(zbuf_v, agg_sh.at[pl.ds(sid * ZROWS + i * ZBUF, ZBUF)])
        return 0
    lax.fori_loop(0, ZROWS // ZBUF, zcopy_body, 0)
    plsc.subcore_barrier()

    ebase = cid * (E // NC) + sid * EPW

    def chunk_body(i, _):
        base = ebase + i * CHUNK
        pltpu.sync_copy(src_hbm.at[pl.ds(base, CHUNK)], src_v)
        pltpu.sync_copy(dst_hbm.at[pl.ds(base, CHUNK)], dst_v)
        pltpu.async_copy(h_hbm.at[src_v], rows_v, sem).wait()
        pltpu.sync_copy(rows_v, agg_sh.at[dst_v], add=True)
        return 0
    lax.fori_loop(0, NCHUNK, chunk_body, 0)
    plsc.subcore_barrier()

    pltpu.sync_copy(agg_sh.at[pl.ds(sid * ZROWS, ZROWS)],
                    out_hbm.at[cid].at[pl.ds(sid * ZROWS, ZROWS)])


@functools.cache
def _get_segsum():
    return pl.kernel(
        _segsum_body,
        out_type=jax.ShapeDtypeStruct((NC, NPAD, D), jnp.float32),
        mesh=plsc.VectorSubcoreMesh(core_axis_name="c", subcore_axis_name="s",
                                    num_cores=NC, num_subcores=NS),
        scratch_types=[
            pltpu.VMEM((CHUNK,), jnp.int32),
            pltpu.VMEM((CHUNK,), jnp.int32),
            pltpu.VMEM((CHUNK, D), jnp.float32),
            pltpu.VMEM((ZBUF, D), jnp.float32),
            pltpu.VMEM_SHARED((NPAD, D), jnp.float32),
            pltpu.SemaphoreType.DMA,
        ],
    )


def _segsum(h, src, dst):
    return _get_segsum()(h, src, dst)


def _mlp_block(h, p0, p1, W1, b1, g, be, W2, b2):
    z = h + p0[:N] + p1[:N]
    u = jnp.dot(z, W1, preferred_element_type=jnp.float32) + b1
    mean = jnp.mean(u, axis=0, keepdims=True)
    var = jnp.mean(jnp.square(u - mean), axis=0, keepdims=True)
    u = (u - mean) / jnp.sqrt(var + 1e-5) * g + be
    u = jnp.maximum(u, 0.0)
    v = jnp.dot(u, W2, preferred_element_type=jnp.float32) + b2
    return jnp.maximum(v, 0.0)


def _tc_layer_body(h_ref, p_ref, W1_ref, b1_ref, g_ref, be_ref, W2_ref,
                   b2_ref, o_ref):
    o_ref[...] = _mlp_block(h_ref[...], p_ref[0], p_ref[1], W1_ref[...],
                            b1_ref[...], g_ref[...], be_ref[...], W2_ref[...],
                            b2_ref[...])


_tc_layer = pl.pallas_call(
    _tc_layer_body,
    out_shape=jax.ShapeDtypeStruct((N, H), jnp.float32),
)


def _tc_final_body(h_ref, p_ref, W1_ref, b1_ref, g_ref, be_ref, W2_ref,
                   b2_ref, batch_ref, mW1_ref, mb1_ref, mW2_ref, mb2_ref,
                   o_ref):
    h3 = _mlp_block(h_ref[...], p_ref[0], p_ref[1], W1_ref[...], b1_ref[...],
                    g_ref[...], be_ref[...], W2_ref[...], b2_ref[...])
    onehot = (batch_ref[...] == lax.broadcasted_iota(jnp.int32, (N, B), 1))
    onehot = onehot.astype(jnp.float32)
    pooled = lax.dot_general(onehot, h3, (((0,), (0,)), ((), ())),
                             preferred_element_type=jnp.float32)
    t = jnp.maximum(
        jnp.dot(pooled, mW1_ref[...], preferred_element_type=jnp.float32)
        + mb1_ref[...], 0.0)
    o_ref[...] = (jnp.dot(t, mW2_ref[...], preferred_element_type=jnp.float32)
                  + mb2_ref[...])


_tc_final = pl.pallas_call(
    _tc_final_body,
    out_shape=jax.ShapeDtypeStruct((B, O), jnp.float32),
)


@jax.jit
def _run(x, edge_index, batch, W1, b1, gamma, beta, W2, b2, mW1, mb1, mW2,
         mb2):
    src = edge_index[0]
    dst = edge_index[1]
    batch2d = batch.reshape(N, 1)
    h = x
    for i in range(L - 1):
        p = _segsum(h, src, dst)
        h = _tc_layer(h, p, W1[i], b1[i].reshape(1, H), gamma[i].reshape(1, H),
                      beta[i].reshape(1, H), W2[i], b2[i].reshape(1, H))
    p = _segsum(h, src, dst)
    i = L - 1
    return _tc_final(h, p, W1[i], b1[i].reshape(1, H), gamma[i].reshape(1, H),
                     beta[i].reshape(1, H), W2[i], b2[i].reshape(1, H),
                     batch2d, mW1, mb1.reshape(1, H), mW2, mb2.reshape(1, O))


def kernel(x, edge_index, batch, batch_size, W1, b1, gamma, beta, W2, b2,
           mW1, mb1, mW2, mb2):
    return _run(x, edge_index, batch, W1, b1, gamma, beta, W2, b2, mW1, mb1,
                mW2, mb2)
